# baseline (speedup 1.0000x reference)
import jax
import jax.numpy as jnp
from jax import lax
from jax.experimental import pallas as pl
from jax.experimental.pallas import tpu as pltpu
from jax.experimental.pallas import tpu_sc as plsc

_POOL = 7
_B, _H, _W, _C = 4, 50, 50, 256
_R = 300
_NC, _NS = 2, 16
_NW = _NC * _NS
_NROI = _B * _R
_RPW = -(-_NROI // _NW)
_NROI_PAD = _NW * _RPW
_WPAD = 40
_CHROWS = 4
_NCH = _C // 16
_NSLOT = 2
_DIST = 1


def _roi_meta(rois):
    r = rois.reshape(_NROI, 4)
    h0 = (_H * r[:, 0]).astype(jnp.int32)
    w0 = (_W * r[:, 1]).astype(jnp.int32)
    h1 = (_H * r[:, 2]).astype(jnp.int32)
    w1 = (_W * r[:, 3]).astype(jnp.int32)
    rh = h1 - h0
    rw = w1 - w0
    hstep = rh // _POOL
    wstep = rw // _POOL
    b = jnp.repeat(jnp.arange(_B, dtype=jnp.int32), _R)
    wbase = jnp.minimum((w0 // 8) * 8, _W - _WPAD - 2)
    dw = w0 - wbase
    nr6 = rh - (_POOL - 1) * hstep
    nchunks = (_POOL - 1) + (nr6 + _CHROWS - 1) // _CHROWS
    meta = jnp.stack([b, h0, wbase, dw, hstep, wstep, rh, rw, nr6, nchunks]
                     + [jnp.zeros_like(h0)] * 6, axis=1)
    pad = jnp.tile(meta[:1], (_NROI_PAD - _NROI, 1))
    return jnp.concatenate([meta, pad], axis=0).reshape(_NW, _RPW * 16)


def _sc_body(fm, meta_hbm, out_hbm, meta_v, rowbuf, outbuf, dsem, osem):
    c = lax.axis_index("c")
    s = lax.axis_index("s")
    wid = c * _NS + s
    pltpu.sync_copy(meta_hbm.at[wid], meta_v)
    ninf = jnp.full((16,), -jnp.inf, jnp.float32)

    def chunk_rows(tt, k):
        mv = meta_v[pl.ds(tt * 16, 16)]
        bin_i = jnp.minimum(k, _POOL - 1)
        p = jnp.maximum(k - (_POOL - 1), 0)
        y0 = mv[1] + bin_i * mv[4] + p * _CHROWS
        nrk = jnp.where(k >= _POOL - 1,
                        jnp.minimum(_CHROWS, mv[8] - p * _CHROWS), mv[4])
        return mv[0], pl.multiple_of(mv[2], 8), y0, nrk

    def issue_chunk(tt, k, slot):
        b, wbase, y0, nrk = chunk_rows(tt, k)

        def issue_row(r, cc):
            pltpu.async_copy(fm.at[b, y0 + r, pl.ds(wbase, _WPAD), :],
                             rowbuf.at[slot, r], dsem.at[slot])
            return cc

        lax.fori_loop(0, nrk, issue_row, 0)

    def drain_chunk(tt, k, slot):
        b, wbase, y0, nrk = chunk_rows(tt, k)

        def drain_row(r, cc):
            pltpu.make_async_copy(fm.at[b, y0 + r, pl.ds(wbase, _WPAD), :],
                                  rowbuf.at[slot, r], dsem.at[slot]).wait()
            return cc

        lax.fori_loop(0, nrk, drain_row, 0)

    def issue_ahead(t, kk, cbase, nchunks):
        slot = lax.rem(cbase + kk, _NSLOT)

        @pl.when(kk < nchunks)
        def _():
            issue_chunk(t, kk, slot)

        @pl.when((kk >= nchunks) & (t + 1 < _RPW))
        def _():
            issue_chunk(t + 1, kk - nchunks, slot)

    for kk in range(_DIST):
        issue_chunk(jnp.int32(0), jnp.int32(kk), jnp.int32(kk))

    def bins_static(slot, nrk, ws, rw, dw, store):
        for j in range(_POOL - 1):
            c0 = dw + j * ws

            def row_loop(r, a):
                out = a
                for w in range(ws):
                    out = tuple(
                        jnp.maximum(out[ch],
                                    rowbuf[slot, r, c0 + w,
                                           pl.ds(ch * 16, 16)])
                        for ch in range(_NCH))
                return out

            init = store.load(j)
            accs = lax.fori_loop(0, nrk, row_loop, init)
            store.save(j, accs)

        c6 = dw + (_POOL - 1) * ws

        def col_loop(w, a):
            def row_loop(r, aa):
                return tuple(
                    jnp.maximum(aa[ch],
                                rowbuf[slot, r, c6 + w, pl.ds(ch * 16, 16)])
                    for ch in range(_NCH))
            return lax.fori_loop(0, nrk, row_loop, a)

        init = store.load(_POOL - 1)
        accs = lax.fori_loop(0, rw - (_POOL - 1) * ws, col_loop, init)
        store.save(_POOL - 1, accs)

    class _FreshStore:
        def __init__(self, pout, bin_i):
            self.pout, self.bin_i = pout, bin_i

        def load(self, j):
            return tuple(ninf for _ in range(_NCH))

        def save(self, j, accs):
            for ch in range(_NCH):
                outbuf[self.pout, self.bin_i, j, pl.ds(ch * 16, 16)] = accs[ch]

    class _AccumStore(_FreshStore):
        def load(self, j):
            return tuple(outbuf[self.pout, self.bin_i, j, pl.ds(ch * 16, 16)]
                         for ch in range(_NCH))

    def compute_chunk(slot, nrk, wstep, rw, dw, store):
        lax.switch(wstep - 2,
                   [lambda: bins_static(slot, nrk, 2, rw, dw, store),
                    lambda: bins_static(slot, nrk, 3, rw, dw, store),
                    lambda: bins_static(slot, nrk, 4, rw, dw, store)])

    def do_roi(t, cbase):
        mv = meta_v[pl.ds(t * 16, 16)]
        dw = mv[3]
        hstep = mv[4]
        wstep = mv[5]
        rw = mv[7]
        nchunks = mv[9]
        pout = lax.rem(t, 2)

        @pl.when(t >= 2)
        def _():
            pltpu.make_async_copy(outbuf.at[pout],
                                  out_hbm.at[wid * _RPW + t - 2],
                                  osem.at[pout]).wait()

        def main_chunk(k, cc):
            slot = lax.rem(cbase + k, _NSLOT)
            issue_ahead(t, k + _DIST, cbase, nchunks)
            drain_chunk(t, k, slot)
            compute_chunk(slot, hstep, wstep, rw, dw, _FreshStore(pout, k))
            return cc

        lax.fori_loop(0, _POOL - 1, main_chunk, 0)

        for j in range(_POOL):
            for ch in range(_NCH):
                outbuf[pout, _POOL - 1, j, pl.ds(ch * 16, 16)] = ninf

        def piece_chunk(k, cc):
            slot = lax.rem(cbase + k, _NSLOT)
            issue_ahead(t, k + _DIST, cbase, nchunks)
            _, _, _, nrk = chunk_rows(t, k)
            drain_chunk(t, k, slot)
            compute_chunk(slot, nrk, wstep, rw, dw,
                          _AccumStore(pout, _POOL - 1))
            return cc

        lax.fori_loop(_POOL - 1, nchunks, piece_chunk, 0)

        pltpu.async_copy(outbuf.at[pout], out_hbm.at[wid * _RPW + t],
                         osem.at[pout])
        return cbase + nchunks

    lax.fori_loop(0, _RPW, do_roi, jnp.int32(0))

    def final_drain(t, cc):
        pout = lax.rem(t, 2)
        pltpu.make_async_copy(outbuf.at[pout], out_hbm.at[wid * _RPW + t],
                              osem.at[pout]).wait()
        return cc

    lax.fori_loop(_RPW - 2, _RPW, final_drain, 0)


def kernel(feature_map, rois):
    meta = _roi_meta(rois)
    mesh = plsc.VectorSubcoreMesh(core_axis_name="c", subcore_axis_name="s")
    run = pl.kernel(
        _sc_body,
        mesh=mesh,
        out_type=jax.ShapeDtypeStruct((_NROI_PAD, _POOL, _POOL, _C),
                                      jnp.float32),
        scratch_types=[
            pltpu.VMEM((_RPW * 16,), jnp.int32),
            pltpu.VMEM((_NSLOT, _CHROWS, _WPAD, _C), jnp.float32),
            pltpu.VMEM((2, _POOL, _POOL, _C), jnp.float32),
            pltpu.SemaphoreType.DMA((_NSLOT,)),
            pltpu.SemaphoreType.DMA((2,)),
        ],
    )
    out = run(feature_map, meta)
    return out[:_NROI].reshape(_B, _R, _POOL, _POOL, _C)


# speedup vs baseline: 1.3515x; 1.3515x over previous
import jax
import jax.numpy as jnp
from jax import lax
from jax.experimental import pallas as pl
from jax.experimental.pallas import tpu as pltpu
from jax.experimental.pallas import tpu_sc as plsc

_POOL = 7
_B, _H, _W, _C = 4, 50, 50, 256
_R = 300
_NC, _NS = 2, 16
_NW = _NC * _NS
_NROI = _B * _R
_RPW = -(-_NROI // _NW)
_NROI_PAD = _NW * _RPW
_WPAD = 40
_CHROWS = 4
_NCH = _C // 16
_NSLOT = 2
_DIST = 1


def _roi_meta(rois):
    r = rois.reshape(_NROI, 4)
    h0 = (_H * r[:, 0]).astype(jnp.int32)
    w0 = (_W * r[:, 1]).astype(jnp.int32)
    h1 = (_H * r[:, 2]).astype(jnp.int32)
    w1 = (_W * r[:, 3]).astype(jnp.int32)
    rh = h1 - h0
    rw = w1 - w0
    hstep = rh // _POOL
    wstep = rw // _POOL
    b = jnp.repeat(jnp.arange(_B, dtype=jnp.int32), _R)
    wbase = jnp.minimum((w0 // 8) * 8, _W - _WPAD - 2)
    dw = w0 - wbase
    nr6 = rh - (_POOL - 1) * hstep
    nchunks = (_POOL - 1) + (nr6 + _CHROWS - 1) // _CHROWS
    meta = jnp.stack([b, h0, wbase, dw, hstep, wstep, rh, rw, nr6, nchunks]
                     + [jnp.zeros_like(h0)] * 6, axis=1)
    pad = jnp.tile(meta[:1], (_NROI_PAD - _NROI, 1))
    return jnp.concatenate([meta, pad], axis=0).reshape(_NW, _RPW * 16)


def _sc_body(fm, meta_hbm, out_hbm, meta_v, rowbuf, outbuf, dsem, osem):
    c = lax.axis_index("c")
    s = lax.axis_index("s")
    wid = c * _NS + s
    pltpu.sync_copy(meta_hbm.at[wid], meta_v)
    ninf = jnp.full((16,), -jnp.inf, jnp.float32)

    def chunk_rows(tt, k):
        mv = meta_v[pl.ds(tt * 16, 16)]
        bin_i = jnp.minimum(k, _POOL - 1)
        p = jnp.maximum(k - (_POOL - 1), 0)
        y0 = mv[1] + bin_i * mv[4] + p * _CHROWS
        nrk = jnp.where(k >= _POOL - 1,
                        jnp.minimum(_CHROWS, mv[8] - p * _CHROWS), mv[4])
        return mv[0], pl.multiple_of(mv[2], 8), y0, nrk

    def issue_chunk(tt, k, slot):
        b, wbase, y0, nrk = chunk_rows(tt, k)

        def issue_row(r, cc):
            pltpu.async_copy(fm.at[b, y0 + r, pl.ds(wbase, _WPAD), :],
                             rowbuf.at[slot, r], dsem.at[slot])
            return cc

        lax.fori_loop(0, nrk, issue_row, 0)

    def drain_chunk(tt, k, slot):
        b, wbase, y0, nrk = chunk_rows(tt, k)

        def drain_row(r, cc):
            pltpu.make_async_copy(fm.at[b, y0 + r, pl.ds(wbase, _WPAD), :],
                                  rowbuf.at[slot, r], dsem.at[slot]).wait()
            return cc

        lax.fori_loop(0, nrk, drain_row, 0)

    def issue_ahead(t, kk, cbase, nchunks):
        slot = lax.rem(cbase + kk, _NSLOT)

        @pl.when(kk < nchunks)
        def _():
            issue_chunk(t, kk, slot)

        @pl.when((kk >= nchunks) & (t + 1 < _RPW))
        def _():
            issue_chunk(t + 1, kk - nchunks, slot)

    for kk in range(_DIST):
        issue_chunk(jnp.int32(0), jnp.int32(kk), jnp.int32(kk))

    def bins_static(slot, nrk, ws, rw, dw, store):
        for j in range(_POOL - 1):
            c0 = dw + j * ws

            def row_loop(r, a):
                out = a
                for w in range(ws):
                    out = tuple(
                        jnp.maximum(out[ch],
                                    rowbuf[slot, r, c0 + w,
                                           pl.ds(ch * 16, 16)])
                        for ch in range(_NCH))
                return out

            init = store.load(j)
            accs = lax.fori_loop(0, nrk, row_loop, init)
            store.save(j, accs)

        c6 = dw + (_POOL - 1) * ws

        def col_loop(w, a):
            def row_loop(r, aa):
                return tuple(
                    jnp.maximum(aa[ch],
                                rowbuf[slot, r, c6 + w, pl.ds(ch * 16, 16)])
                    for ch in range(_NCH))
            return lax.fori_loop(0, nrk, row_loop, a)

        init = store.load(_POOL - 1)
        accs = lax.fori_loop(0, rw - (_POOL - 1) * ws, col_loop, init)
        store.save(_POOL - 1, accs)

    class _FreshStore:
        def __init__(self, pout, bin_i):
            self.pout, self.bin_i = pout, bin_i

        def load(self, j):
            return tuple(ninf for _ in range(_NCH))

        def save(self, j, accs):
            for ch in range(_NCH):
                outbuf[self.pout, self.bin_i, j, pl.ds(ch * 16, 16)] = accs[ch]

    class _AccumStore(_FreshStore):
        def load(self, j):
            return tuple(outbuf[self.pout, self.bin_i, j, pl.ds(ch * 16, 16)]
                         for ch in range(_NCH))

    def compute_chunk(slot, nrk, wstep, rw, dw, store):
        lax.switch(wstep - 2,
                   [lambda: bins_static(slot, nrk, 2, rw, dw, store),
                    lambda: bins_static(slot, nrk, 3, rw, dw, store),
                    lambda: bins_static(slot, nrk, 4, rw, dw, store)])

    def do_roi(t, cbase):
        mv = meta_v[pl.ds(t * 16, 16)]
        dw = mv[3]
        hstep = mv[4]
        wstep = mv[5]
        rw = mv[7]
        nchunks = mv[9]
        pout = lax.rem(t, 2)

        @pl.when((t >= 2) & (wid * _RPW + t - 2 < _NROI))
        def _():
            ps = wid * _RPW + t - 2
            pltpu.make_async_copy(outbuf.at[pout],
                                  out_hbm.at[ps // _R, lax.rem(ps, _R)],
                                  osem.at[pout]).wait()

        def main_chunk(k, cc):
            slot = lax.rem(cbase + k, _NSLOT)
            issue_ahead(t, k + _DIST, cbase, nchunks)
            drain_chunk(t, k, slot)
            compute_chunk(slot, hstep, wstep, rw, dw, _FreshStore(pout, k))
            return cc

        lax.fori_loop(0, _POOL - 1, main_chunk, 0)

        for j in range(_POOL):
            for ch in range(_NCH):
                outbuf[pout, _POOL - 1, j, pl.ds(ch * 16, 16)] = ninf

        def piece_chunk(k, cc):
            slot = lax.rem(cbase + k, _NSLOT)
            issue_ahead(t, k + _DIST, cbase, nchunks)
            _, _, _, nrk = chunk_rows(t, k)
            drain_chunk(t, k, slot)
            compute_chunk(slot, nrk, wstep, rw, dw,
                          _AccumStore(pout, _POOL - 1))
            return cc

        lax.fori_loop(_POOL - 1, nchunks, piece_chunk, 0)

        slot = wid * _RPW + t

        @pl.when(slot < _NROI)
        def _():
            pltpu.async_copy(outbuf.at[pout],
                             out_hbm.at[slot // _R, lax.rem(slot, _R)],
                             osem.at[pout])

        return cbase + nchunks

    lax.fori_loop(0, _RPW, do_roi, jnp.int32(0))

    def final_drain(t, cc):
        pout = lax.rem(t, 2)
        slot = wid * _RPW + t

        @pl.when(slot < _NROI)
        def _():
            pltpu.make_async_copy(outbuf.at[pout],
                                  out_hbm.at[slot // _R, lax.rem(slot, _R)],
                                  osem.at[pout]).wait()
        return cc

    lax.fori_loop(_RPW - 2, _RPW, final_drain, 0)


def kernel(feature_map, rois):
    meta = _roi_meta(rois)
    mesh = plsc.VectorSubcoreMesh(core_axis_name="c", subcore_axis_name="s")
    run = pl.kernel(
        _sc_body,
        mesh=mesh,
        out_type=jax.ShapeDtypeStruct((_B, _R, _POOL, _POOL, _C),
                                      jnp.float32),
        scratch_types=[
            pltpu.VMEM((_RPW * 16,), jnp.int32),
            pltpu.VMEM((_NSLOT, _CHROWS, _WPAD, _C), jnp.float32),
            pltpu.VMEM((2, _POOL, _POOL, _C), jnp.float32),
            pltpu.SemaphoreType.DMA((_NSLOT,)),
            pltpu.SemaphoreType.DMA((2,)),
        ],
    )
    return run(feature_map, meta)
